# baseline (speedup 1.0000x reference)
import functools

import jax
import jax.numpy as jnp
from jax import lax
from jax.experimental import pallas as pl
from jax.experimental.pallas import tpu as pltpu
from jax.experimental.pallas import tpu_sc as plsc

N = 16384
V = 100000
VP = 100352
D = 32
K = 8
H = 64

QB = 16
VT = 512
NT = VP // VT
NP = 4
TP = NT // NP
NB = 3
CW = NP * NB * VT

BIG = 3.0e38
PAD_COORD = 1.0e4
IBIG = 2 ** 30


def _knn_body(x_ref, vt_ref, idx_ref, cd_ref, ci_ref):
    x0 = x_ref[:, 0:1]
    x1 = x_ref[:, 1:2]
    x2 = x_ref[:, 2:3]
    xsq = x_ref[:, 3:4]
    lane = jax.lax.broadcasted_iota(jnp.int32, (QB, VT), 1)

    for p in range(NP):
        m1 = jnp.full((QB, VT), BIG, jnp.float32)
        m2 = m1
        m3 = m1
        i1 = jnp.zeros((QB, VT), jnp.int32)
        i2 = i1
        i3 = i1

        def tile(t, carry, p=p):
            m1, m2, m3, i1, i2, i3 = carry
            off = pl.multiple_of(p * (TP * VT) + t * VT, VT)
            v = vt_ref[:, pl.ds(off, VT)]
            dot = (x0 * v[0:1, :] + x1 * v[1:2, :]) + x2 * v[2:3, :]
            d = (xsq - 2.0 * dot) + v[3:4, :]
            ii = lane + off
            c1 = d < m1
            c2 = d < m2
            c3 = d < m3
            m3n = jnp.where(c2, m2, jnp.where(c3, d, m3))
            i3n = jnp.where(c2, i2, jnp.where(c3, ii, i3))
            m2n = jnp.where(c1, m1, jnp.where(c2, d, m2))
            i2n = jnp.where(c1, i1, jnp.where(c2, ii, i2))
            m1n = jnp.minimum(d, m1)
            i1n = jnp.where(c1, ii, i1)
            return m1n, m2n, m3n, i1n, i2n, i3n

        m1, m2, m3, i1, i2, i3 = jax.lax.fori_loop(
            0, TP, tile, (m1, m2, m3, i1, i2, i3))
        base = p * (NB * VT)
        cd_ref[:, base:base + VT] = m1
        cd_ref[:, base + VT:base + 2 * VT] = m2
        cd_ref[:, base + 2 * VT:base + 3 * VT] = m3
        ci_ref[:, base:base + VT] = i1
        ci_ref[:, base + VT:base + 2 * VT] = i2
        ci_ref[:, base + 2 * VT:base + 3 * VT] = i3

    ci = ci_ref[:, :]
    for k in range(K):
        d = cd_ref[:, :]
        m = jnp.min(d, axis=1, keepdims=True)
        eq = d == m
        am = jnp.min(jnp.where(eq, ci, IBIG), axis=1, keepdims=True)
        idx_ref[:, k] = am[:, 0]
        cd_ref[:, :] = jnp.where(eq & (ci == am), BIG, d)


def _round_bf16(x):
    u = jax.lax.bitcast_convert_type(x, jnp.uint32)
    u = (u + jnp.uint32(0x7FFF) + ((u >> 16) & jnp.uint32(1))) & jnp.uint32(0xFFFF0000)
    return jax.lax.bitcast_convert_type(u, jnp.float32)


def _sqnorm(a):
    return (a[:, 0] * a[:, 0] + a[:, 2] * a[:, 2]) + a[:, 1] * a[:, 1]


@functools.partial(jax.jit, static_argnums=())
def _knn(xyz, mesh_vertices):
    vpad = jnp.pad(mesh_vertices, ((0, VP - V), (0, 0)),
                   constant_values=PAD_COORD)
    vt = jnp.concatenate([_round_bf16(vpad), _sqnorm(vpad)[:, None]],
                         axis=1).T
    xq = jnp.concatenate([_round_bf16(xyz), _sqnorm(xyz)[:, None]],
                         axis=1)
    idx = pl.pallas_call(
        _knn_body,
        grid=(N // QB,),
        in_specs=[
            pl.BlockSpec((QB, 4), lambda i: (i, 0)),
            pl.BlockSpec((4, VP), lambda i: (0, 0)),
        ],
        out_specs=pl.BlockSpec((QB, K), lambda i: (i, 0)),
        out_shape=jax.ShapeDtypeStruct((N, K), jnp.int32),
        scratch_shapes=[pltpu.VMEM((QB, CW), jnp.float32),
                        pltpu.VMEM((QB, CW), jnp.int32)],
    )(xq, vt)
    return idx


NIDX = N * K
NWORK = 32
PERW = NIDX // NWORK
GCH = 128


def _gather_body(table_hbm, idx_hbm, out_hbm, idx_v, rows_v, sem):
    wid = lax.axis_index("s") * 2 + lax.axis_index("c")
    for c in range(PERW // GCH):
        base = wid * PERW + c * GCH
        pltpu.sync_copy(idx_hbm.at[pl.ds(base, GCH)], idx_v)
        pltpu.async_copy(table_hbm.at[idx_v], rows_v, sem).wait()
        pltpu.sync_copy(rows_v, out_hbm.at[pl.ds(base, GCH)])


def _sc_gather(table, idx_flat):
    mesh = plsc.VectorSubcoreMesh(core_axis_name="c", subcore_axis_name="s")
    f = functools.partial(
        pl.kernel,
        mesh=mesh,
        out_type=jax.ShapeDtypeStruct((NIDX, 128), jnp.float32),
        scratch_types=[
            pltpu.VMEM((GCH,), jnp.int32),
            pltpu.VMEM((GCH, 128), jnp.float32),
            pltpu.SemaphoreType.DMA,
        ],
    )(_gather_body)
    return f(table, idx_flat)


QB2 = 512


def _blend_body(g_ref, x_ref, vd_ref, w1_ref, b1_ref, w2_ref, b2_ref,
                ws1_ref, bs1_ref, ws2_ref, bs2_ref, wg_ref, bg_ref,
                sdf_ref, col_ref):
    g = g_ref[...]
    x = x_ref[...]
    vd = vd_ref[...]
    nx = g[:, :, 96:99]
    mk = g[:, :, 99]
    diff = x[:, None, :] - nx
    ds = jnp.sqrt(jnp.sum(diff * diff, axis=-1))
    w = 1.0 / (ds + 1e-8)
    w = w / jnp.sum(w, axis=-1, keepdims=True)
    wk = w[..., None]
    nabla = jnp.sum(wk * diff, axis=-2)
    nabla = nabla / (jnp.sqrt(jnp.sum(nabla * nabla, axis=-1, keepdims=True)) + 1e-8)
    feat = jnp.sum(wk * g[:, :, 0:32], axis=-2)
    geo = jnp.sum(wk * g[:, :, 64:96], axis=-2)
    sdf_ref[...] = (jnp.sum(_round_bf16(geo) * wg_ref[...], axis=-1,
                            keepdims=True) + bg_ref[...])
    hp = jax.lax.Precision.HIGHEST
    x1 = _round_bf16(jnp.concatenate([feat, vd, nabla], axis=-1))
    h = jax.nn.relu(jnp.dot(x1, w1_ref[...], precision=hp) + b1_ref[...])
    colors = jax.nn.sigmoid(jnp.dot(_round_bf16(h), w2_ref[...], precision=hp)
                            + b2_ref[...])
    paint = jnp.sum(mk, axis=-1, keepdims=True) >= K
    sw = w * mk
    sw = sw / (jnp.sum(sw, axis=-1, keepdims=True) + 1e-8)
    sfeat = jnp.sum(sw[..., None] * g[:, :, 32:64], axis=-2)
    x2 = _round_bf16(jnp.concatenate([sfeat, vd, nabla], axis=-1))
    hs = jax.nn.relu(jnp.dot(x2, ws1_ref[...], precision=hp) + bs1_ref[...])
    slave = jax.nn.sigmoid(jnp.dot(_round_bf16(hs), ws2_ref[...], precision=hp)
                           + bs2_ref[...])
    col_ref[...] = jnp.where(paint, slave, colors)


def _blend(gath, xyz, view_dirs, W1, b1, W2, b2, Ws1, bs1, Ws2, bs2, Wg, bg):
    din = D + 6
    row = lambda a: a.reshape(1, -1)
    full = lambda r, c: pl.BlockSpec((r, c), lambda i: (0, 0))
    sdf, col = pl.pallas_call(
        _blend_body,
        grid=(N // QB2,),
        in_specs=[
            pl.BlockSpec((QB2, K, 128), lambda i: (i, 0, 0)),
            pl.BlockSpec((QB2, 3), lambda i: (i, 0)),
            pl.BlockSpec((QB2, 3), lambda i: (i, 0)),
            full(din, H), full(1, H), full(H, 3), full(1, 3),
            full(din, H), full(1, H), full(H, 3), full(1, 3),
            full(1, D), full(1, 1),
        ],
        out_specs=[pl.BlockSpec((QB2, 1), lambda i: (i, 0)),
                   pl.BlockSpec((QB2, 3), lambda i: (i, 0))],
        out_shape=[jax.ShapeDtypeStruct((N, 1), jnp.float32),
                   jax.ShapeDtypeStruct((N, 3), jnp.float32)],
    )(gath, xyz, view_dirs,
      _round_bf16(W1), row(b1), _round_bf16(W2), row(b2),
      _round_bf16(Ws1), row(bs1), _round_bf16(Ws2), row(bs2),
      _round_bf16(Wg).T, bg.reshape(1, 1))
    return sdf[:, 0], col


def kernel(xyz, view_dirs, mesh_vertices, color_features, edit_color_features,
           geo_features, main_mask, W1, b1, W2, b2, Ws1, bs1, Ws2, bs2, Wg, bg):
    idx = _knn(xyz, mesh_vertices)
    table = jnp.concatenate(
        [color_features, edit_color_features, geo_features, mesh_vertices,
         main_mask[:, None].astype(jnp.float32),
         jnp.zeros((V, 128 - 3 * D - 4), jnp.float32)], axis=1)
    gath = _sc_gather(table, idx.reshape(-1)).reshape(N, K, 128)
    return _blend(gath, xyz, view_dirs,
                  W1, b1, W2, b2, Ws1, bs1, Ws2, bs2, Wg, bg)


# speedup vs baseline: 2.2668x; 2.2668x over previous
import functools

import jax
import jax.numpy as jnp
from jax import lax
from jax.experimental import pallas as pl
from jax.experimental.pallas import tpu as pltpu
from jax.experimental.pallas import tpu_sc as plsc

N = 16384
V = 100000
VP = 100352
D = 32
K = 8
H = 64

QB = 16
VT = 512
NT = VP // VT
NP = 4
TP = NT // NP
NB = 3
UNR = 7
CW = NP * NB * VT

BIG = 3.0e38
PAD_COORD = 1.0e4
IBIG = 2 ** 30


def _knn_body(x_ref, vt_ref, idx_ref, cd_ref, ci_ref):
    x0 = x_ref[:, 0:1]
    x1 = x_ref[:, 1:2]
    x2 = x_ref[:, 2:3]
    xsq = x_ref[:, 3:4]
    lane = jax.lax.broadcasted_iota(jnp.int32, (QB, VT), 1)

    for p in range(NP):
        m1 = jnp.full((QB, VT), BIG, jnp.float32)
        m2 = m1
        m3 = m1
        i1 = jnp.zeros((QB, VT), jnp.int32)
        i2 = i1
        i3 = i1

        def tile(t, carry, p=p):
            m1, m2, m3, i1, i2, i3 = carry
            for u in range(UNR):
                off = pl.multiple_of(p * (TP * VT) + (t * UNR + u) * VT, VT)
                v = vt_ref[:, pl.ds(off, VT)]
                dot = (x0 * v[0:1, :] + x1 * v[1:2, :]) + x2 * v[2:3, :]
                d = (xsq - 2.0 * dot) + v[3:4, :]
                ii = lane + off
                c1 = d < m1
                c2 = d < m2
                c3 = d < m3
                m3n = jnp.where(c2, m2, jnp.where(c3, d, m3))
                i3n = jnp.where(c2, i2, jnp.where(c3, ii, i3))
                m2n = jnp.where(c1, m1, jnp.where(c2, d, m2))
                i2n = jnp.where(c1, i1, jnp.where(c2, ii, i2))
                m1, m2, m3 = jnp.minimum(d, m1), m2n, m3n
                i1, i2, i3 = jnp.where(c1, ii, i1), i2n, i3n
            return m1, m2, m3, i1, i2, i3

        m1, m2, m3, i1, i2, i3 = jax.lax.fori_loop(
            0, TP // UNR, tile, (m1, m2, m3, i1, i2, i3))
        base = p * (NB * VT)
        cd_ref[:, base:base + VT] = m1
        cd_ref[:, base + VT:base + 2 * VT] = m2
        cd_ref[:, base + 2 * VT:base + 3 * VT] = m3
        ci_ref[:, base:base + VT] = i1
        ci_ref[:, base + VT:base + 2 * VT] = i2
        ci_ref[:, base + 2 * VT:base + 3 * VT] = i3

    ci = ci_ref[:, :]
    for k in range(K):
        d = cd_ref[:, :]
        m = jnp.min(d, axis=1, keepdims=True)
        eq = d == m
        am = jnp.min(jnp.where(eq, ci, IBIG), axis=1, keepdims=True)
        idx_ref[:, k] = am[:, 0]
        cd_ref[:, :] = jnp.where(eq & (ci == am), BIG, d)


def _round_bf16(x):
    u = jax.lax.bitcast_convert_type(x, jnp.uint32)
    u = (u + jnp.uint32(0x7FFF) + ((u >> 16) & jnp.uint32(1))) & jnp.uint32(0xFFFF0000)
    return jax.lax.bitcast_convert_type(u, jnp.float32)


def _sqnorm(a):
    return (a[:, 0] * a[:, 0] + a[:, 2] * a[:, 2]) + a[:, 1] * a[:, 1]


@functools.partial(jax.jit, static_argnums=())
def _knn(xyz, mesh_vertices):
    vpad = jnp.pad(mesh_vertices, ((0, VP - V), (0, 0)),
                   constant_values=PAD_COORD)
    vt = jnp.concatenate([_round_bf16(vpad), _sqnorm(vpad)[:, None]],
                         axis=1).T
    xq = jnp.concatenate([_round_bf16(xyz), _sqnorm(xyz)[:, None]],
                         axis=1)
    idx = pl.pallas_call(
        _knn_body,
        grid=(N // QB,),
        in_specs=[
            pl.BlockSpec((QB, 4), lambda i: (i, 0)),
            pl.BlockSpec((4, VP), lambda i: (0, 0)),
        ],
        out_specs=pl.BlockSpec((QB, K), lambda i: (i, 0)),
        out_shape=jax.ShapeDtypeStruct((N, K), jnp.int32),
        scratch_shapes=[pltpu.VMEM((QB, CW), jnp.float32),
                        pltpu.VMEM((QB, CW), jnp.int32)],
    )(xq, vt)
    return idx


NIDX = N * K
NWORK = 32
PERW = NIDX // NWORK
GCH = 128


def _gather_body(table_hbm, idx_hbm, out_hbm, idx_v, rows_v, sem):
    wid = lax.axis_index("s") * 2 + lax.axis_index("c")
    for c in range(PERW // GCH):
        base = wid * PERW + c * GCH
        pltpu.sync_copy(idx_hbm.at[pl.ds(base, GCH)], idx_v)
        pltpu.async_copy(table_hbm.at[idx_v], rows_v, sem).wait()
        pltpu.sync_copy(rows_v, out_hbm.at[pl.ds(base, GCH)])


def _sc_gather(table, idx_flat):
    mesh = plsc.VectorSubcoreMesh(core_axis_name="c", subcore_axis_name="s")
    f = functools.partial(
        pl.kernel,
        mesh=mesh,
        out_type=jax.ShapeDtypeStruct((NIDX, 128), jnp.float32),
        scratch_types=[
            pltpu.VMEM((GCH,), jnp.int32),
            pltpu.VMEM((GCH, 128), jnp.float32),
            pltpu.SemaphoreType.DMA,
        ],
    )(_gather_body)
    return f(table, idx_flat)


QB2 = 512


def _blend_body(g_ref, x_ref, vd_ref, w1_ref, b1_ref, w2_ref, b2_ref,
                ws1_ref, bs1_ref, ws2_ref, bs2_ref, wg_ref, bg_ref,
                sdf_ref, col_ref):
    g = g_ref[...]
    x = x_ref[...]
    vd = vd_ref[...]
    nx = g[:, :, 96:99]
    mk = g[:, :, 99]
    diff = x[:, None, :] - nx
    ds = jnp.sqrt(jnp.sum(diff * diff, axis=-1))
    w = 1.0 / (ds + 1e-8)
    w = w / jnp.sum(w, axis=-1, keepdims=True)
    wk = w[..., None]
    nabla = jnp.sum(wk * diff, axis=-2)
    nabla = nabla / (jnp.sqrt(jnp.sum(nabla * nabla, axis=-1, keepdims=True)) + 1e-8)
    feat = jnp.sum(wk * g[:, :, 0:32], axis=-2)
    geo = jnp.sum(wk * g[:, :, 64:96], axis=-2)
    sdf_ref[...] = (jnp.sum(_round_bf16(geo) * wg_ref[...], axis=-1,
                            keepdims=True) + bg_ref[...])
    hp = jax.lax.Precision.HIGHEST
    x1 = _round_bf16(jnp.concatenate([feat, vd, nabla], axis=-1))
    h = jax.nn.relu(jnp.dot(x1, w1_ref[...], precision=hp) + b1_ref[...])
    colors = jax.nn.sigmoid(jnp.dot(_round_bf16(h), w2_ref[...], precision=hp)
                            + b2_ref[...])
    paint = jnp.sum(mk, axis=-1, keepdims=True) >= K
    sw = w * mk
    sw = sw / (jnp.sum(sw, axis=-1, keepdims=True) + 1e-8)
    sfeat = jnp.sum(sw[..., None] * g[:, :, 32:64], axis=-2)
    x2 = _round_bf16(jnp.concatenate([sfeat, vd, nabla], axis=-1))
    hs = jax.nn.relu(jnp.dot(x2, ws1_ref[...], precision=hp) + bs1_ref[...])
    slave = jax.nn.sigmoid(jnp.dot(_round_bf16(hs), ws2_ref[...], precision=hp)
                           + bs2_ref[...])
    col_ref[...] = jnp.where(paint, slave, colors)


def _blend(gath, xyz, view_dirs, W1, b1, W2, b2, Ws1, bs1, Ws2, bs2, Wg, bg):
    din = D + 6
    row = lambda a: a.reshape(1, -1)
    full = lambda r, c: pl.BlockSpec((r, c), lambda i: (0, 0))
    sdf, col = pl.pallas_call(
        _blend_body,
        grid=(N // QB2,),
        in_specs=[
            pl.BlockSpec((QB2, K, 128), lambda i: (i, 0, 0)),
            pl.BlockSpec((QB2, 3), lambda i: (i, 0)),
            pl.BlockSpec((QB2, 3), lambda i: (i, 0)),
            full(din, H), full(1, H), full(H, 3), full(1, 3),
            full(din, H), full(1, H), full(H, 3), full(1, 3),
            full(1, D), full(1, 1),
        ],
        out_specs=[pl.BlockSpec((QB2, 1), lambda i: (i, 0)),
                   pl.BlockSpec((QB2, 3), lambda i: (i, 0))],
        out_shape=[jax.ShapeDtypeStruct((N, 1), jnp.float32),
                   jax.ShapeDtypeStruct((N, 3), jnp.float32)],
    )(gath, xyz, view_dirs,
      _round_bf16(W1), row(b1), _round_bf16(W2), row(b2),
      _round_bf16(Ws1), row(bs1), _round_bf16(Ws2), row(bs2),
      _round_bf16(Wg).T, bg.reshape(1, 1))
    return sdf[:, 0], col


def kernel(xyz, view_dirs, mesh_vertices, color_features, edit_color_features,
           geo_features, main_mask, W1, b1, W2, b2, Ws1, bs1, Ws2, bs2, Wg, bg):
    idx = _knn(xyz, mesh_vertices)
    table = jnp.concatenate(
        [color_features, edit_color_features, geo_features, mesh_vertices,
         main_mask[:, None].astype(jnp.float32),
         jnp.zeros((V, 128 - 3 * D - 4), jnp.float32)], axis=1)
    gath = _sc_gather(table, idx.reshape(-1)).reshape(N, K, 128)
    return _blend(gath, xyz, view_dirs,
                  W1, b1, W2, b2, Ws1, bs1, Ws2, bs2, Wg, bg)


# speedup vs baseline: 2.7848x; 1.2285x over previous
import functools

import jax
import jax.numpy as jnp
from jax import lax
from jax.experimental import pallas as pl
from jax.experimental.pallas import tpu as pltpu
from jax.experimental.pallas import tpu_sc as plsc

N = 16384
V = 100000
VP = 100352
D = 32
K = 8
H = 64

QB = 16
VT = 512
NT = VP // VT
NP = 4
TP = NT // NP
NB = 3
UNR = 49
CW = NP * NB * VT

BIG = 3.0e38
PAD_COORD = 1.0e4
IBIG = 2 ** 30


def _knn_body(x_ref, vt_ref, idx_ref, cd_ref, ci_ref):
    x0 = x_ref[:, 0:1]
    x1 = x_ref[:, 1:2]
    x2 = x_ref[:, 2:3]
    xsq = x_ref[:, 3:4]
    lane = jax.lax.broadcasted_iota(jnp.int32, (QB, VT), 1)

    for p in range(NP):
        m1 = jnp.full((QB, VT), BIG, jnp.float32)
        m2 = m1
        m3 = m1
        i1 = jnp.zeros((QB, VT), jnp.int32)
        i2 = i1
        i3 = i1

        def tile(t, carry, p=p):
            m1, m2, m3, i1, i2, i3 = carry
            for u in range(UNR):
                off = pl.multiple_of(p * (TP * VT) + (t * UNR + u) * VT, VT)
                v = vt_ref[:, pl.ds(off, VT)]
                dot = (x0 * v[0:1, :] + x1 * v[1:2, :]) + x2 * v[2:3, :]
                d = (xsq - 2.0 * dot) + v[3:4, :]
                ii = lane + off
                c1 = d < m1
                c2 = d < m2
                c3 = d < m3
                m3n = jnp.where(c2, m2, jnp.where(c3, d, m3))
                i3n = jnp.where(c2, i2, jnp.where(c3, ii, i3))
                m2n = jnp.where(c1, m1, jnp.where(c2, d, m2))
                i2n = jnp.where(c1, i1, jnp.where(c2, ii, i2))
                m1, m2, m3 = jnp.minimum(d, m1), m2n, m3n
                i1, i2, i3 = jnp.where(c1, ii, i1), i2n, i3n
            return m1, m2, m3, i1, i2, i3

        m1, m2, m3, i1, i2, i3 = jax.lax.fori_loop(
            0, TP // UNR, tile, (m1, m2, m3, i1, i2, i3))
        base = p * (NB * VT)
        cd_ref[:, base:base + VT] = m1
        cd_ref[:, base + VT:base + 2 * VT] = m2
        cd_ref[:, base + 2 * VT:base + 3 * VT] = m3
        ci_ref[:, base:base + VT] = i1
        ci_ref[:, base + VT:base + 2 * VT] = i2
        ci_ref[:, base + 2 * VT:base + 3 * VT] = i3

    ci = ci_ref[:, :]
    for k in range(K):
        d = cd_ref[:, :]
        m = jnp.min(d, axis=1, keepdims=True)
        eq = d == m
        am = jnp.min(jnp.where(eq, ci, IBIG), axis=1, keepdims=True)
        idx_ref[:, k] = am[:, 0]
        cd_ref[:, :] = jnp.where(eq & (ci == am), BIG, d)


def _round_bf16(x):
    u = jax.lax.bitcast_convert_type(x, jnp.uint32)
    u = (u + jnp.uint32(0x7FFF) + ((u >> 16) & jnp.uint32(1))) & jnp.uint32(0xFFFF0000)
    return jax.lax.bitcast_convert_type(u, jnp.float32)


def _sqnorm(a):
    return (a[:, 0] * a[:, 0] + a[:, 2] * a[:, 2]) + a[:, 1] * a[:, 1]


@functools.partial(jax.jit, static_argnums=())
def _knn(xyz, mesh_vertices):
    vpad = jnp.pad(mesh_vertices, ((0, VP - V), (0, 0)),
                   constant_values=PAD_COORD)
    vt = jnp.concatenate([_round_bf16(vpad), _sqnorm(vpad)[:, None]],
                         axis=1).T
    xq = jnp.concatenate([_round_bf16(xyz), _sqnorm(xyz)[:, None]],
                         axis=1)
    idx = pl.pallas_call(
        _knn_body,
        grid=(N // QB,),
        in_specs=[
            pl.BlockSpec((QB, 4), lambda i: (i, 0)),
            pl.BlockSpec((4, VP), lambda i: (0, 0)),
        ],
        out_specs=pl.BlockSpec((QB, K), lambda i: (i, 0)),
        out_shape=jax.ShapeDtypeStruct((N, K), jnp.int32),
        scratch_shapes=[pltpu.VMEM((QB, CW), jnp.float32),
                        pltpu.VMEM((QB, CW), jnp.int32)],
    )(xq, vt)
    return idx


NIDX = N * K
NWORK = 32
PERW = NIDX // NWORK
GCH = 128


def _gather_body(table_hbm, idx_hbm, out_hbm, idx_v, rows_v, sem):
    wid = lax.axis_index("s") * 2 + lax.axis_index("c")
    for c in range(PERW // GCH):
        base = wid * PERW + c * GCH
        pltpu.sync_copy(idx_hbm.at[pl.ds(base, GCH)], idx_v)
        pltpu.async_copy(table_hbm.at[idx_v], rows_v, sem).wait()
        pltpu.sync_copy(rows_v, out_hbm.at[pl.ds(base, GCH)])


def _sc_gather(table, idx_flat):
    mesh = plsc.VectorSubcoreMesh(core_axis_name="c", subcore_axis_name="s")
    f = functools.partial(
        pl.kernel,
        mesh=mesh,
        out_type=jax.ShapeDtypeStruct((NIDX, 128), jnp.float32),
        scratch_types=[
            pltpu.VMEM((GCH,), jnp.int32),
            pltpu.VMEM((GCH, 128), jnp.float32),
            pltpu.SemaphoreType.DMA,
        ],
    )(_gather_body)
    return f(table, idx_flat)


QB2 = 512


def _blend_body(g_ref, x_ref, vd_ref, w1_ref, b1_ref, w2_ref, b2_ref,
                ws1_ref, bs1_ref, ws2_ref, bs2_ref, wg_ref, bg_ref,
                sdf_ref, col_ref):
    g = g_ref[...]
    x = x_ref[...]
    vd = vd_ref[...]
    nx = g[:, :, 96:99]
    mk = g[:, :, 99]
    diff = x[:, None, :] - nx
    ds = jnp.sqrt(jnp.sum(diff * diff, axis=-1))
    w = 1.0 / (ds + 1e-8)
    w = w / jnp.sum(w, axis=-1, keepdims=True)
    wk = w[..., None]
    nabla = jnp.sum(wk * diff, axis=-2)
    nabla = nabla / (jnp.sqrt(jnp.sum(nabla * nabla, axis=-1, keepdims=True)) + 1e-8)
    feat = jnp.sum(wk * g[:, :, 0:32], axis=-2)
    geo = jnp.sum(wk * g[:, :, 64:96], axis=-2)
    sdf_ref[...] = (jnp.sum(_round_bf16(geo) * wg_ref[...], axis=-1,
                            keepdims=True) + bg_ref[...])
    hp = jax.lax.Precision.HIGHEST
    x1 = _round_bf16(jnp.concatenate([feat, vd, nabla], axis=-1))
    h = jax.nn.relu(jnp.dot(x1, w1_ref[...], precision=hp) + b1_ref[...])
    colors = jax.nn.sigmoid(jnp.dot(_round_bf16(h), w2_ref[...], precision=hp)
                            + b2_ref[...])
    paint = jnp.sum(mk, axis=-1, keepdims=True) >= K
    sw = w * mk
    sw = sw / (jnp.sum(sw, axis=-1, keepdims=True) + 1e-8)
    sfeat = jnp.sum(sw[..., None] * g[:, :, 32:64], axis=-2)
    x2 = _round_bf16(jnp.concatenate([sfeat, vd, nabla], axis=-1))
    hs = jax.nn.relu(jnp.dot(x2, ws1_ref[...], precision=hp) + bs1_ref[...])
    slave = jax.nn.sigmoid(jnp.dot(_round_bf16(hs), ws2_ref[...], precision=hp)
                           + bs2_ref[...])
    col_ref[...] = jnp.where(paint, slave, colors)


def _blend(gath, xyz, view_dirs, W1, b1, W2, b2, Ws1, bs1, Ws2, bs2, Wg, bg):
    din = D + 6
    row = lambda a: a.reshape(1, -1)
    full = lambda r, c: pl.BlockSpec((r, c), lambda i: (0, 0))
    sdf, col = pl.pallas_call(
        _blend_body,
        grid=(N // QB2,),
        in_specs=[
            pl.BlockSpec((QB2, K, 128), lambda i: (i, 0, 0)),
            pl.BlockSpec((QB2, 3), lambda i: (i, 0)),
            pl.BlockSpec((QB2, 3), lambda i: (i, 0)),
            full(din, H), full(1, H), full(H, 3), full(1, 3),
            full(din, H), full(1, H), full(H, 3), full(1, 3),
            full(1, D), full(1, 1),
        ],
        out_specs=[pl.BlockSpec((QB2, 1), lambda i: (i, 0)),
                   pl.BlockSpec((QB2, 3), lambda i: (i, 0))],
        out_shape=[jax.ShapeDtypeStruct((N, 1), jnp.float32),
                   jax.ShapeDtypeStruct((N, 3), jnp.float32)],
    )(gath, xyz, view_dirs,
      _round_bf16(W1), row(b1), _round_bf16(W2), row(b2),
      _round_bf16(Ws1), row(bs1), _round_bf16(Ws2), row(bs2),
      _round_bf16(Wg).T, bg.reshape(1, 1))
    return sdf[:, 0], col


def kernel(xyz, view_dirs, mesh_vertices, color_features, edit_color_features,
           geo_features, main_mask, W1, b1, W2, b2, Ws1, bs1, Ws2, bs2, Wg, bg):
    idx = _knn(xyz, mesh_vertices)
    table = jnp.concatenate(
        [color_features, edit_color_features, geo_features, mesh_vertices,
         main_mask[:, None].astype(jnp.float32),
         jnp.zeros((V, 128 - 3 * D - 4), jnp.float32)], axis=1)
    gath = _sc_gather(table, idx.reshape(-1)).reshape(N, K, 128)
    return _blend(gath, xyz, view_dirs,
                  W1, b1, W2, b2, Ws1, bs1, Ws2, bs2, Wg, bg)
